# baseline (speedup 1.0000x reference)
import functools

import jax
import jax.numpy as jnp
from jax import lax
from jax.experimental import pallas as pl
from jax.experimental.pallas import tpu as pltpu
from jax.experimental.pallas import tpu_sc as plsc

N = 50000
NTS = 9
L = 32
E = 300000
NW = 32
CHUNK = 9600
P = NW * CHUNK
KB = 128
NK = CHUNK // KB
NK2 = 2 * NK
BF = 15
TAB = N * (N + 1) // 2
TSIZE = TAB + 8
DUMMY = N
JW = 80
JW1 = 64
SUB = 5
RB = SUB * KB
NBIG = NK // SUB
SGR = 5
EPT = 2 * P // 16
NKS = EPT // KB
NBIGS = NKS // SGR
HALF = N // 2
ACC_ROWS = 25024
STRIPE = ACC_ROWS // 16


def _wid():
    return lax.axis_index("s") * 2 + lax.axis_index("c")


def _lazy(builder):
    cache = []

    def call(*args):
        if not cache:
            cache.append(builder())
        return cache[0](*args)

    return call


def _sc_params():
    return pltpu.CompilerParams(use_tc_tiling_on_sc=False)


def _vmesh():
    return plsc.VectorSubcoreMesh(core_axis_name="c", subcore_axis_name="s")


def _build_sc_scatter_ids():
  @functools.partial(
      pl.kernel,
      out_type=jax.ShapeDtypeStruct((TSIZE,), jnp.int32),
      mesh=_vmesh(),
      name="sc_dedup_scatter",
      compiler_params=_sc_params(),
      scratch_types=[
          pltpu.VMEM((NK, KB), jnp.int32),
          pltpu.VMEM((NK, KB), jnp.int32),
          pltpu.SemaphoreType.DMA,
      ],
  )
  def _sc_scatter_ids(t_hbm, ids_hbm, table_hbm, idx_v, val_v, sem):
    w = _wid()
    pltpu.sync_copy(t_hbm.at[w], idx_v)
    pltpu.sync_copy(ids_hbm.at[w], val_v)

    def fire(j, c):
        pltpu.async_copy(val_v.at[j], table_hbm.at[idx_v.at[j]], sem)
        return c

    def drain(j, c):
        pltpu.make_async_copy(val_v.at[0], table_hbm.at[idx_v.at[0]],
                              sem).wait()
        return c

    def batch(b, c):
        lax.fori_loop(b * BF, (b + 1) * BF, fire, 0)
        lax.fori_loop(0, BF, drain, 0)
        return c

    lax.fori_loop(0, NK // BF, batch, 0)

  return _sc_scatter_ids


_sc_scatter_ids = _lazy(_build_sc_scatter_ids)


def _build_sc_mark_reps():
  @functools.partial(
      pl.kernel,
      out_type=[
          jax.ShapeDtypeStruct((NW, NK, KB), jnp.int32),
          jax.ShapeDtypeStruct((NW, NK, KB), jnp.int32),
      ],
      mesh=_vmesh(),
      name="sc_dedup_mark",
      compiler_params=_sc_params(),
      scratch_types=[
          pltpu.VMEM((NK, KB), jnp.int32),
          pltpu.VMEM((NK, KB), jnp.int32),
          pltpu.VMEM((NK, KB), jnp.int32),
          pltpu.VMEM((NK, KB), jnp.int32),
          pltpu.VMEM((NK, KB), jnp.int32),
          pltpu.VMEM((NK, KB), jnp.int32),
          pltpu.SemaphoreType.DMA,
      ],
  )
  def _sc_mark_reps(t_hbm, lo_hbm, hi_hbm, table_hbm, r1_hbm, r2_hbm,
                    idx_v, w_v, lo_v, hi_v, r1_v, r2_v, sem):
    w = _wid()
    pltpu.sync_copy(t_hbm.at[w], idx_v)
    pltpu.sync_copy(lo_hbm.at[w], lo_v)
    pltpu.sync_copy(hi_hbm.at[w], hi_v)

    def fire(j, c):
        pltpu.async_copy(table_hbm.at[idx_v.at[j]], w_v.at[j], sem)
        return c

    def drain(j, c):
        pltpu.make_async_copy(table_hbm.at[idx_v.at[0]], w_v.at[0],
                              sem).wait()
        return c

    def batch(b, c):
        lax.fori_loop(b * BF, (b + 1) * BF, fire, 0)
        lax.fori_loop(0, BF, drain, 0)
        return c

    lax.fori_loop(0, NK // BF, batch, 0)

    base = w * CHUNK
    lanes = lax.broadcasted_iota(jnp.int32, (16,), 0)

    def cbody(i, carry):
        j = i // 8
        o = (i % 8) * 16
        wv = w_v[j, pl.ds(o, 16)]
        lov = lo_v[j, pl.ds(o, 16)]
        hiv = hi_v[j, pl.ds(o, 16)]
        pos = base + i * 16 + lanes
        m = (wv == pos) & (pos < E)
        r1_v[j, pl.ds(o, 16)] = jnp.where(m, hiv, DUMMY)
        r2_v[j, pl.ds(o, 16)] = jnp.where(m, lov, DUMMY)
        return carry

    lax.fori_loop(0, CHUNK // 16, cbody, 0)
    pltpu.sync_copy(r1_v, r1_hbm.at[w])
    pltpu.sync_copy(r2_v, r2_hbm.at[w])

  return _sc_mark_reps


_sc_mark_reps = _lazy(_build_sc_mark_reps)


def _make_sc_gather(width, name):
    @functools.partial(
        pl.kernel,
        out_type=jax.ShapeDtypeStruct((2, P, width), jnp.float32),
        mesh=_vmesh(),
        name=name,
        compiler_params=_sc_params(),
        scratch_types=[
            pltpu.VMEM((NK2, KB), jnp.int32),
            pltpu.VMEM((2, RB, width), jnp.float32),
            pltpu.SemaphoreType.DMA,
            pltpu.SemaphoreType.DMA,
        ],
    )
    def _sc_gather(j_hbm, idx_hbm, out_hbm, idx_v, buf, semg, semw):
        w = _wid()
        pltpu.sync_copy(idx_hbm.at[w], idx_v)

        def body(g, c):
            slot = g % 2

            @pl.when(g >= 2)
            def _():
                pltpu.make_async_copy(
                    buf.at[0], out_hbm.at[0].at[pl.ds(0, RB)], semw).wait()

            h = g // NBIG
            b = g - h * NBIG
            for s in range(SUB):
                pltpu.async_copy(
                    j_hbm.at[idx_v.at[h * NK + b * SUB + s]],
                    buf.at[slot].at[pl.ds(s * KB, KB)], semg)
            for s in range(SUB):
                pltpu.make_async_copy(
                    j_hbm.at[idx_v.at[0]],
                    buf.at[0].at[pl.ds(0, KB)], semg).wait()
            row0 = (w * NK + b * SUB) * KB
            pltpu.async_copy(buf.at[slot],
                             out_hbm.at[h].at[pl.ds(row0, RB)], semw)
            return c

        lax.fori_loop(0, 2 * NBIG, body, 0)
        for _ in range(2):
            pltpu.make_async_copy(
                buf.at[0], out_hbm.at[0].at[pl.ds(0, RB)], semw).wait()

    return _sc_gather


_sc_gather_j0 = _lazy(lambda: _make_sc_gather(JW, "sc_gather_j0"))
_sc_gather_j1 = _lazy(lambda: _make_sc_gather(JW1, "sc_gather_j1"))


def _build_sc_segsum():
  @functools.partial(
      pl.kernel,
      out_type=jax.ShapeDtypeStruct((2, ACC_ROWS, L), jnp.float32),
      mesh=_vmesh(),
      name="sc_segsum",
      compiler_params=_sc_params(),
      scratch_types=[
          pltpu.VMEM((NKS, KB), jnp.int32),
          pltpu.VMEM((2, SGR * KB, L), jnp.float32),
          pltpu.VMEM_SHARED((ACC_ROWS, L), jnp.float32),
          pltpu.SemaphoreType.DMA,
      ],
  )
  def _sc_segsum(he_hbm, recv_hbm, zeros_hbm, agg_hbm, idx_v, buf, acc, seml):
    c = lax.axis_index("c")
    s = lax.axis_index("s")
    pltpu.sync_copy(recv_hbm.at[c].at[s], idx_v)
    pltpu.sync_copy(zeros_hbm.at[pl.ds(s * STRIPE, STRIPE)],
                    acc.at[pl.ds(s * STRIPE, STRIPE)])
    plsc.subcore_barrier()

    eb = s * EPT
    last = 2 * P - SGR * KB

    def load(q, slot):
        row0 = jnp.minimum(eb + q * SGR * KB, last)
        pltpu.async_copy(he_hbm.at[pl.ds(row0, SGR * KB)], buf.at[slot], seml)

    def wait_load():
        pltpu.make_async_copy(he_hbm.at[pl.ds(0, SGR * KB)], buf.at[0],
                              seml).wait()

    def adds(q, slot):
        for u in range(SGR):
            pltpu.sync_copy(buf.at[slot].at[pl.ds(u * KB, KB)],
                            acc.at[idx_v.at[q * SGR + u]], add=True)

    load(0, 0)

    def body(qq, cc):
        wait_load()
        load(2 * qq + 1, 1)
        adds(2 * qq, 0)
        wait_load()
        load(2 * qq + 2, 0)
        adds(2 * qq + 1, 1)
        return cc

    lax.fori_loop(0, NBIGS // 2, body, 0)
    wait_load()
    plsc.subcore_barrier()
    pltpu.sync_copy(acc.at[pl.ds(s * STRIPE, STRIPE)],
                    agg_hbm.at[c].at[pl.ds(s * STRIPE, STRIPE)])

  return _sc_segsum


_sc_segsum = _lazy(_build_sc_segsum)


RN = 1000
RE = 2048


def _tc1_body(nt_ref, vel_ref, mp_ref, nmean, nstd, enW1, enb1, enW2, enb2,
              Wa, Wb, h_ref, j_ref):
    nt = nt_ref[...]
    oh = (lax.broadcasted_iota(jnp.int32, (RN, NTS), 1) == nt).astype(jnp.float32)
    nf = jnp.concatenate([vel_ref[...], oh], axis=1)
    nfn = (nf - nmean[...]) / nstd[...]
    h = jnp.dot(jax.nn.relu(jnp.dot(nfn, enW1[...]) + enb1[...]),
                enW2[...]) + enb2[...]
    h_ref[...] = h
    z = jnp.zeros((RN, JW - 66), jnp.float32)
    j_ref[...] = jnp.concatenate(
        [jnp.dot(h, Wa[...]), jnp.dot(h, Wb[...]), mp_ref[...], z], axis=1)


def _tc2_body(jlo_ref, jhi_ref, emean, estd, eeW1, eeb1, eeW2, eeb2,
              Wc, b1, W2, b2, he_ref):
    jlo = jlo_ref[0]
    jhi = jhi_ref[0]
    rel = jlo[:, 64:66] - jhi[:, 64:66]
    nrm = jnp.sqrt(jnp.sum(rel * rel, axis=1, keepdims=True))
    for d in (0, 1):
        r = rel if d == 0 else -rel
        ef = jnp.concatenate([r, nrm], axis=1)
        efn = (ef - emean[...]) / estd[...]
        he0 = jnp.dot(jax.nn.relu(jnp.dot(efn, eeW1[...]) + eeb1[...]),
                      eeW2[...]) + eeb2[...]
        if d == 0:
            pre = jlo[:, 0:32] + jhi[:, 32:64] + jnp.dot(he0, Wc[...]) + b1[...]
        else:
            pre = jhi[:, 0:32] + jlo[:, 32:64] + jnp.dot(he0, Wc[...]) + b1[...]
        he_ref[d] = he0 + jnp.dot(jax.nn.relu(pre), W2[...]) + b2[...]


def _tc3_body(h_ref, agg_ref, Wh, Wg, bn1, Wn2, bn2, Wa, Wb, h1_ref, j_ref):
    h = h_ref[...]
    agg = agg_ref[0]
    pre = jnp.dot(h, Wh[...]) + jnp.dot(agg, Wg[...]) + bn1[...]
    h1 = h + jnp.dot(jax.nn.relu(pre), Wn2[...]) + bn2[...]
    h1_ref[...] = h1
    j_ref[...] = jnp.concatenate([jnp.dot(h1, Wa[...]), jnp.dot(h1, Wb[...])],
                                 axis=1)


def _tc4_body(jlo_ref, jhi_ref, he_ref, Wc, b1, W2, b2, heo_ref):
    jlo = jlo_ref[0]
    jhi = jhi_ref[0]
    for d in (0, 1):
        he = he_ref[d]
        if d == 0:
            pre = jlo[:, 0:32] + jhi[:, 32:64] + jnp.dot(he, Wc[...]) + b1[...]
        else:
            pre = jhi[:, 0:32] + jlo[:, 32:64] + jnp.dot(he, Wc[...]) + b1[...]
        heo_ref[d] = he + jnp.dot(jax.nn.relu(pre), W2[...]) + b2[...]


def _tc5_body(h_ref, agg_ref, vel_ref, Wh, Wg, bn1, Wn2, bn2,
              dW1, db1, dW2, db2, omean, ostd, out_ref, upd_ref):
    h = h_ref[...]
    agg = agg_ref[0]
    pre = jnp.dot(h, Wh[...]) + jnp.dot(agg, Wg[...]) + bn1[...]
    h2 = h + jnp.dot(jax.nn.relu(pre), Wn2[...]) + bn2[...]
    o = jnp.dot(jax.nn.relu(jnp.dot(h2, dW1[...]) + db1[...]), dW2[...]) + db2[...]
    out_ref[...] = o
    upd_ref[...] = vel_ref[...] + o * ostd[...] + omean[...]


def _full(shape):
    return pl.BlockSpec(shape, lambda i: tuple(0 for _ in shape))


def _rows(shape):
    return pl.BlockSpec(shape, lambda i: (i,) + tuple(0 for _ in shape[1:]))


def _rows3(shape):
    return pl.BlockSpec(shape, lambda i: (0, i, 0))


def _half3(shape, h):
    return pl.BlockSpec(shape, lambda i, _h=h: (_h, i, 0))


def _aggspec():
    return pl.BlockSpec((1, RN, L), lambda i: (i // (HALF // RN),
                                               i % (HALF // RN), 0))


def _tc_call(body, grid, in_specs, out_specs, out_shape):
    return pl.pallas_call(body, grid=(grid,), in_specs=in_specs,
                          out_specs=out_specs, out_shape=out_shape)


def _r2(v):
    return v.reshape(1, -1)


def kernel(node_type, velocity, mesh_pos, cells, is_training, params):
    p = params
    f32 = jnp.float32

    e = jnp.concatenate([cells[:, 0:2], cells[:, 1:3],
                         jnp.stack([cells[:, 2], cells[:, 0]], axis=1)], axis=0)
    lo = jnp.minimum(e[:, 0], e[:, 1])
    hi = jnp.maximum(e[:, 0], e[:, 1])
    tri = jnp.where(lo % 2 == 0, (lo // 2) * (lo + 1), lo * ((lo + 1) // 2))
    t = lo * N - tri + hi
    pad = P - E
    t_pad = jnp.concatenate([t, jnp.full((pad,), TAB, jnp.int32)])
    lo_pad = jnp.concatenate([lo, jnp.zeros((pad,), jnp.int32)])
    hi_pad = jnp.concatenate([hi, jnp.zeros((pad,), jnp.int32)])
    ids = jnp.arange(P, dtype=jnp.int32)

    t3 = t_pad.reshape(NW, NK, KB)
    ids3 = ids.reshape(NW, NK, KB)
    lo3 = lo_pad.reshape(NW, NK, KB)
    hi3 = hi_pad.reshape(NW, NK, KB)
    lohi = jnp.concatenate([lo3, hi3], axis=1)

    table = _sc_scatter_ids(t3, ids3)
    r1, r2 = _sc_mark_reps(t3, lo3, hi3, table)
    recvg = jnp.concatenate([r1.reshape(P), r2.reshape(P)])
    lc0 = recvg
    lc1 = recvg - HALF
    recvs = jnp.stack([
        jnp.where(lc0 < HALF, lc0, HALF).reshape(16, NKS, KB),
        jnp.where((lc1 >= 0) & (lc1 < HALF), lc1, HALF).reshape(16, NKS, KB)])

    w0 = p['We1'][0]
    h0, j0 = _tc_call(
        _tc1_body, N // RN,
        [_rows((RN, 1)), _rows((RN, 2)), _rows((RN, 2)),
         _full((1, NTS + 2)), _full((1, NTS + 2)),
         _full((NTS + 2, L)), _full((1, L)), _full((L, L)), _full((1, L)),
         _full((L, L)), _full((L, L))],
        [_rows((RN, L)), _rows((RN, JW))],
        [jax.ShapeDtypeStruct((N, L), f32), jax.ShapeDtypeStruct((N, JW), f32)],
    )(node_type, velocity, mesh_pos, _r2(p['node_mean']), _r2(p['node_std']),
      p['enW1'], _r2(p['enb1']), p['enW2'], _r2(p['enb2']),
      w0[0:L], w0[L:2 * L])

    jg0 = _sc_gather_j0(j0, lohi)

    he1 = _tc_call(
        _tc2_body, P // RE,
        [_half3((1, RE, JW), 0), _half3((1, RE, JW), 1),
         _full((1, 3)), _full((1, 3)), _full((3, L)), _full((1, L)),
         _full((L, L)), _full((1, L)), _full((L, L)), _full((1, L)),
         _full((L, L)), _full((1, L))],
        _rows3((2, RE, L)),
        jax.ShapeDtypeStruct((2, P, L), f32),
    )(jg0, jg0, _r2(p['edge_mean']), _r2(p['edge_std']), p['eeW1'],
      _r2(p['eeb1']), p['eeW2'], _r2(p['eeb2']), w0[2 * L:3 * L],
      _r2(p['be1'][0]), p['We2'][0], _r2(p['be2'][0]))

    zeros = jnp.zeros((ACC_ROWS, L), f32)

    agg0 = _sc_segsum(he1.reshape(2 * P, L), recvs, zeros)

    wn0 = p['Wn1'][0]
    w1 = p['We1'][1]
    h1, j1 = _tc_call(
        _tc3_body, N // RN,
        [_rows((RN, L)), _aggspec(),
         _full((L, L)), _full((L, L)), _full((1, L)), _full((L, L)),
         _full((1, L)), _full((L, L)), _full((L, L))],
        [_rows((RN, L)), _rows((RN, JW1))],
        [jax.ShapeDtypeStruct((N, L), f32), jax.ShapeDtypeStruct((N, JW1), f32)],
    )(h0, agg0, wn0[0:L], wn0[L:2 * L], _r2(p['bn1'][0]), p['Wn2'][0],
      _r2(p['bn2'][0]), w1[0:L], w1[L:2 * L])

    jg1 = _sc_gather_j1(j1, lohi)

    he2 = _tc_call(
        _tc4_body, P // RE,
        [_half3((1, RE, JW1), 0), _half3((1, RE, JW1), 1), _rows3((2, RE, L)),
         _full((L, L)), _full((1, L)), _full((L, L)), _full((1, L))],
        _rows3((2, RE, L)),
        jax.ShapeDtypeStruct((2, P, L), f32),
    )(jg1, jg1, he1, w1[2 * L:3 * L], _r2(p['be1'][1]), p['We2'][1],
      _r2(p['be2'][1]))

    agg1 = _sc_segsum(he2.reshape(2 * P, L), recvs, zeros)

    wn1 = p['Wn1'][1]
    out, updated = _tc_call(
        _tc5_body, N // RN,
        [_rows((RN, L)), _aggspec(), _rows((RN, 2)),
         _full((L, L)), _full((L, L)), _full((1, L)), _full((L, L)),
         _full((1, L)), _full((L, L)), _full((1, L)), _full((L, 2)),
         _full((1, 2)), _full((1, 2)), _full((1, 2))],
        [_rows((RN, 2)), _rows((RN, 2))],
        [jax.ShapeDtypeStruct((N, 2), f32), jax.ShapeDtypeStruct((N, 2), f32)],
    )(h1, agg1, velocity, wn1[0:L], wn1[L:2 * L], _r2(p['bn1'][1]),
      p['Wn2'][1], _r2(p['bn2'][1]), p['dW1'], _r2(p['db1']), p['dW2'],
      _r2(p['db2']), _r2(p['out_mean']), _r2(p['out_std']))

    return jnp.where(is_training != 0, out, updated)


# speedup vs baseline: 1.1129x; 1.1129x over previous
import functools

import jax
import jax.numpy as jnp
from jax import lax
from jax.experimental import pallas as pl
from jax.experimental.pallas import tpu as pltpu
from jax.experimental.pallas import tpu_sc as plsc

N = 50000
NTS = 9
L = 32
E = 300000
NW = 32
CHUNK = 9600
P = NW * CHUNK
KB = 128
NK = CHUNK // KB
NK2 = 2 * NK
BF = 15
TAB = N * (N + 1) // 2
TSIZE = TAB + 8
DUMMY = N
JW = 80
JW1 = 64
SUB = 5
RB = SUB * KB
NBIG = NK // SUB
SUBJ = 3
SGR = 5
EPT = 2 * P // 16
NKS = EPT // KB
NBIGS = NKS // SGR
HALF = N // 2
ACC_ROWS = 25024
STRIPE = ACC_ROWS // 16


def _wid():
    return lax.axis_index("s") * 2 + lax.axis_index("c")


def _lazy(builder):
    cache = []

    def call(*args):
        if not cache:
            cache.append(builder())
        return cache[0](*args)

    return call


def _sc_params():
    return pltpu.CompilerParams(use_tc_tiling_on_sc=False)


def _vmesh():
    return plsc.VectorSubcoreMesh(core_axis_name="c", subcore_axis_name="s")


def _build_sc_scatter_gather_j0():
  @functools.partial(
      pl.kernel,
      out_type=[
          jax.ShapeDtypeStruct((TSIZE,), jnp.int32),
          jax.ShapeDtypeStruct((2, P, JW), jnp.float32),
      ],
      mesh=_vmesh(),
      name="sc_dedup_scatter_gather_j0",
      compiler_params=_sc_params(),
      scratch_types=[
          pltpu.VMEM((NK, KB), jnp.int32),
          pltpu.VMEM((NK, KB), jnp.int32),
          pltpu.VMEM((NK2, KB), jnp.int32),
          pltpu.VMEM((2, SUBJ * KB, JW), jnp.float32),
          pltpu.SemaphoreType.DMA,
          pltpu.SemaphoreType.DMA,
          pltpu.SemaphoreType.DMA,
      ],
  )
  def _fused(t_hbm, ids_hbm, j_hbm, lohi_hbm, table_hbm, out_hbm,
             tv, val_v, idx_v, buf, semt, semg, semw):
    w = _wid()
    pltpu.sync_copy(t_hbm.at[w], tv)
    pltpu.sync_copy(ids_hbm.at[w], val_v)
    pltpu.sync_copy(lohi_hbm.at[w], idx_v)

    def fire_scat(j, c):
        @pl.when(j < NK)
        def _():
            pltpu.async_copy(val_v.at[j], table_hbm.at[tv.at[j]], semt)
        return c

    def drain_scat(j, c):
        pltpu.make_async_copy(val_v.at[0], table_hbm.at[tv.at[0]],
                              semt).wait()
        return c

    NBJ = NK // SUBJ
    SPG = (NK + 2 * NBJ - 1) // (2 * NBJ)

    def body(g, c):
        slot = g % 2

        @pl.when(g >= 2)
        def _():
            pltpu.make_async_copy(
                buf.at[0], out_hbm.at[0].at[pl.ds(0, SUBJ * KB)], semw).wait()

        lax.fori_loop(g * SPG, (g + 1) * SPG, fire_scat, 0)
        h = g // NBJ
        b = g - h * NBJ
        for u in range(SUBJ):
            pltpu.async_copy(
                j_hbm.at[idx_v.at[h * NK + b * SUBJ + u]],
                buf.at[slot].at[pl.ds(u * KB, KB)], semg)
        for u in range(SUBJ):
            pltpu.make_async_copy(
                j_hbm.at[idx_v.at[0]],
                buf.at[0].at[pl.ds(0, KB)], semg).wait()
        row0 = (w * NK + b * SUBJ) * KB
        pltpu.async_copy(buf.at[slot],
                         out_hbm.at[h].at[pl.ds(row0, SUBJ * KB)], semw)
        return c

    lax.fori_loop(0, 2 * NBJ, body, 0)
    for _ in range(2):
        pltpu.make_async_copy(
            buf.at[0], out_hbm.at[0].at[pl.ds(0, SUBJ * KB)], semw).wait()
    lax.fori_loop(0, NK, drain_scat, 0)

  return _fused


_sc_scatter_gather_j0 = _lazy(_build_sc_scatter_gather_j0)


def _build_sc_mark_reps():
  @functools.partial(
      pl.kernel,
      out_type=[
          jax.ShapeDtypeStruct((NW, NK, KB), jnp.int32),
          jax.ShapeDtypeStruct((NW, NK, KB), jnp.int32),
      ],
      mesh=_vmesh(),
      name="sc_dedup_mark",
      compiler_params=_sc_params(),
      scratch_types=[
          pltpu.VMEM((NK, KB), jnp.int32),
          pltpu.VMEM((NK, KB), jnp.int32),
          pltpu.VMEM((NK, KB), jnp.int32),
          pltpu.VMEM((NK, KB), jnp.int32),
          pltpu.VMEM((NK, KB), jnp.int32),
          pltpu.VMEM((NK, KB), jnp.int32),
          pltpu.SemaphoreType.DMA,
      ],
  )
  def _sc_mark_reps(t_hbm, lo_hbm, hi_hbm, table_hbm, r1_hbm, r2_hbm,
                    idx_v, w_v, lo_v, hi_v, r1_v, r2_v, sem):
    w = _wid()
    pltpu.sync_copy(t_hbm.at[w], idx_v)
    pltpu.sync_copy(lo_hbm.at[w], lo_v)
    pltpu.sync_copy(hi_hbm.at[w], hi_v)

    def fire(j, c):
        pltpu.async_copy(table_hbm.at[idx_v.at[j]], w_v.at[j], sem)
        return c

    def drain(j, c):
        pltpu.make_async_copy(table_hbm.at[idx_v.at[0]], w_v.at[0],
                              sem).wait()
        return c

    def batch(b, c):
        lax.fori_loop(b * BF, (b + 1) * BF, fire, 0)
        lax.fori_loop(0, BF, drain, 0)
        return c

    lax.fori_loop(0, NK // BF, batch, 0)

    base = w * CHUNK
    lanes = lax.broadcasted_iota(jnp.int32, (16,), 0)

    def cbody(i, carry):
        j = i // 8
        o = (i % 8) * 16
        wv = w_v[j, pl.ds(o, 16)]
        lov = lo_v[j, pl.ds(o, 16)]
        hiv = hi_v[j, pl.ds(o, 16)]
        pos = base + i * 16 + lanes
        m = (wv == pos) & (pos < E)
        r1_v[j, pl.ds(o, 16)] = jnp.where(m, hiv, DUMMY)
        r2_v[j, pl.ds(o, 16)] = jnp.where(m, lov, DUMMY)
        return carry

    lax.fori_loop(0, CHUNK // 16, cbody, 0)
    pltpu.sync_copy(r1_v, r1_hbm.at[w])
    pltpu.sync_copy(r2_v, r2_hbm.at[w])

  return _sc_mark_reps


_sc_mark_reps = _lazy(_build_sc_mark_reps)


def _make_sc_gather(width, name):
    @functools.partial(
        pl.kernel,
        out_type=jax.ShapeDtypeStruct((2, P, width), jnp.float32),
        mesh=_vmesh(),
        name=name,
        compiler_params=_sc_params(),
        scratch_types=[
            pltpu.VMEM((NK2, KB), jnp.int32),
            pltpu.VMEM((2, RB, width), jnp.float32),
            pltpu.SemaphoreType.DMA,
            pltpu.SemaphoreType.DMA,
        ],
    )
    def _sc_gather(j_hbm, idx_hbm, out_hbm, idx_v, buf, semg, semw):
        w = _wid()
        pltpu.sync_copy(idx_hbm.at[w], idx_v)

        def body(g, c):
            slot = g % 2

            @pl.when(g >= 2)
            def _():
                pltpu.make_async_copy(
                    buf.at[0], out_hbm.at[0].at[pl.ds(0, RB)], semw).wait()

            h = g // NBIG
            b = g - h * NBIG
            for s in range(SUB):
                pltpu.async_copy(
                    j_hbm.at[idx_v.at[h * NK + b * SUB + s]],
                    buf.at[slot].at[pl.ds(s * KB, KB)], semg)
            for s in range(SUB):
                pltpu.make_async_copy(
                    j_hbm.at[idx_v.at[0]],
                    buf.at[0].at[pl.ds(0, KB)], semg).wait()
            row0 = (w * NK + b * SUB) * KB
            pltpu.async_copy(buf.at[slot],
                             out_hbm.at[h].at[pl.ds(row0, RB)], semw)
            return c

        lax.fori_loop(0, 2 * NBIG, body, 0)
        for _ in range(2):
            pltpu.make_async_copy(
                buf.at[0], out_hbm.at[0].at[pl.ds(0, RB)], semw).wait()

    return _sc_gather


_sc_gather_j1 = _lazy(lambda: _make_sc_gather(JW1, "sc_gather_j1"))


def _build_sc_segsum():
  @functools.partial(
      pl.kernel,
      out_type=jax.ShapeDtypeStruct((2, ACC_ROWS, L), jnp.float32),
      mesh=_vmesh(),
      name="sc_segsum",
      compiler_params=_sc_params(),
      scratch_types=[
          pltpu.VMEM((NKS, KB), jnp.int32),
          pltpu.VMEM((2, SGR * KB, L), jnp.float32),
          pltpu.VMEM_SHARED((ACC_ROWS, L), jnp.float32),
          pltpu.SemaphoreType.DMA,
          pltpu.SemaphoreType.DMA,
          pltpu.SemaphoreType.DMA,
      ],
  )
  def _sc_segsum(he_hbm, recv_hbm, zeros_hbm, agg_hbm, idx_v, buf, acc,
                 seml, sems0, sems1):
    c = lax.axis_index("c")
    s = lax.axis_index("s")
    pltpu.sync_copy(recv_hbm.at[c].at[s], idx_v)
    pltpu.sync_copy(zeros_hbm.at[pl.ds(s * STRIPE, STRIPE)],
                    acc.at[pl.ds(s * STRIPE, STRIPE)])
    plsc.subcore_barrier()

    eb = s * EPT
    last = 2 * P - SGR * KB

    def load(q, slot):
        row0 = jnp.minimum(eb + q * SGR * KB, last)
        pltpu.async_copy(he_hbm.at[pl.ds(row0, SGR * KB)], buf.at[slot], seml)

    def wait_load():
        pltpu.make_async_copy(he_hbm.at[pl.ds(0, SGR * KB)], buf.at[0],
                              seml).wait()

    def adds(q, slot, sem):
        for u in range(SGR):
            pltpu.async_copy(buf.at[slot].at[pl.ds(u * KB, KB)],
                             acc.at[idx_v.at[q * SGR + u]], sem, add=True)

    def drain_adds(slot, sem):
        for u in range(SGR):
            pltpu.make_async_copy(buf.at[slot].at[pl.ds(u * KB, KB)],
                                  acc.at[pl.ds(0, KB)], sem).wait()

    load(0, 0)

    def body(qq, cc):
        wait_load()

        @pl.when(qq >= 1)
        def _():
            drain_adds(1, sems1)

        load(2 * qq + 1, 1)
        adds(2 * qq, 0, sems0)
        wait_load()
        drain_adds(0, sems0)
        load(2 * qq + 2, 0)
        adds(2 * qq + 1, 1, sems1)
        return cc

    lax.fori_loop(0, NBIGS // 2, body, 0)
    wait_load()
    drain_adds(1, sems1)
    plsc.subcore_barrier()
    pltpu.sync_copy(acc.at[pl.ds(s * STRIPE, STRIPE)],
                    agg_hbm.at[c].at[pl.ds(s * STRIPE, STRIPE)])

  return _sc_segsum


_sc_segsum = _lazy(_build_sc_segsum)


RN = 1000
RE = 4096


def _tc1_body(nt_ref, vel_ref, mp_ref, enW1, enb1, enW2, enb2,
              Wa, Wb, h_ref, j_ref):
    nt = nt_ref[...]
    oh = (lax.broadcasted_iota(jnp.int32, (RN, NTS), 1) == nt).astype(jnp.float32)
    nf = jnp.concatenate([vel_ref[...], oh], axis=1)
    h = jnp.dot(jax.nn.relu(jnp.dot(nf, enW1[...]) + enb1[...]),
                enW2[...]) + enb2[...]
    h_ref[...] = h
    z = jnp.zeros((RN, JW - 66), jnp.float32)
    j_ref[...] = jnp.concatenate(
        [jnp.dot(h, Wa[...]), jnp.dot(h, Wb[...]), mp_ref[...], z], axis=1)


def _tc2_body(jlo_ref, jhi_ref, eeWr, eeWn, eeb1, eeW2, eeb2,
              Wc, b1, W2, b2, he_ref):
    jlo = jlo_ref[0]
    jhi = jhi_ref[0]
    rel = jlo[:, 64:66] - jhi[:, 64:66]
    nrm = jnp.sqrt(jnp.sum(rel * rel, axis=1, keepdims=True))
    relW = jnp.dot(rel, eeWr[...])
    base = jnp.dot(nrm, eeWn[...]) + eeb1[...]
    for d in (0, 1):
        he0 = jnp.dot(jax.nn.relu(base + relW if d == 0 else base - relW),
                      eeW2[...]) + eeb2[...]
        if d == 0:
            pre = jlo[:, 0:32] + jhi[:, 32:64] + jnp.dot(he0, Wc[...]) + b1[...]
        else:
            pre = jhi[:, 0:32] + jlo[:, 32:64] + jnp.dot(he0, Wc[...]) + b1[...]
        he_ref[d] = he0 + jnp.dot(jax.nn.relu(pre), W2[...]) + b2[...]


def _tc3_body(h_ref, agg_ref, Wh, Wg, bn1, Wn2, bn2, Wa, Wb, h1_ref, j_ref):
    h = h_ref[...]
    agg = agg_ref[0]
    pre = jnp.dot(h, Wh[...]) + jnp.dot(agg, Wg[...]) + bn1[...]
    h1 = h + jnp.dot(jax.nn.relu(pre), Wn2[...]) + bn2[...]
    h1_ref[...] = h1
    j_ref[...] = jnp.concatenate([jnp.dot(h1, Wa[...]), jnp.dot(h1, Wb[...])],
                                 axis=1)


def _tc4_body(jlo_ref, jhi_ref, he_ref, Wc, b1, W2, b2, heo_ref):
    jlo = jlo_ref[0]
    jhi = jhi_ref[0]
    for d in (0, 1):
        he = he_ref[d]
        if d == 0:
            pre = jlo[:, 0:32] + jhi[:, 32:64] + jnp.dot(he, Wc[...]) + b1[...]
        else:
            pre = jhi[:, 0:32] + jlo[:, 32:64] + jnp.dot(he, Wc[...]) + b1[...]
        heo_ref[d] = he + jnp.dot(jax.nn.relu(pre), W2[...]) + b2[...]


def _tc5_body(h_ref, agg_ref, vel_ref, Wh, Wg, bn1, Wn2, bn2,
              dW1, db1, dW2, db2, omean, ostd, out_ref, upd_ref):
    h = h_ref[...]
    agg = agg_ref[0]
    pre = jnp.dot(h, Wh[...]) + jnp.dot(agg, Wg[...]) + bn1[...]
    h2 = h + jnp.dot(jax.nn.relu(pre), Wn2[...]) + bn2[...]
    o = jnp.dot(jax.nn.relu(jnp.dot(h2, dW1[...]) + db1[...]), dW2[...]) + db2[...]
    out_ref[...] = o
    upd_ref[...] = vel_ref[...] + o * ostd[...] + omean[...]


def _full(shape):
    return pl.BlockSpec(shape, lambda i: tuple(0 for _ in shape))


def _rows(shape):
    return pl.BlockSpec(shape, lambda i: (i,) + tuple(0 for _ in shape[1:]))


def _rows3(shape):
    return pl.BlockSpec(shape, lambda i: (0, i, 0))


def _half3(shape, h):
    return pl.BlockSpec(shape, lambda i, _h=h: (_h, i, 0))


def _aggspec():
    return pl.BlockSpec((1, RN, L), lambda i: (i // (HALF // RN),
                                               i % (HALF // RN), 0))


def _tc_call(body, grid, in_specs, out_specs, out_shape):
    return pl.pallas_call(body, grid=(grid,), in_specs=in_specs,
                          out_specs=out_specs, out_shape=out_shape)


def _r2(v):
    return v.reshape(1, -1)


def kernel(node_type, velocity, mesh_pos, cells, is_training, params):
    p = params
    f32 = jnp.float32

    e = jnp.concatenate([cells[:, 0:2], cells[:, 1:3],
                         jnp.stack([cells[:, 2], cells[:, 0]], axis=1)], axis=0)
    lo = jnp.minimum(e[:, 0], e[:, 1])
    hi = jnp.maximum(e[:, 0], e[:, 1])
    tri = jnp.where(lo % 2 == 0, (lo // 2) * (lo + 1), lo * ((lo + 1) // 2))
    t = lo * N - tri + hi
    pad = P - E
    t_pad = jnp.concatenate([t, jnp.full((pad,), TAB, jnp.int32)])
    lo_pad = jnp.concatenate([lo, jnp.zeros((pad,), jnp.int32)])
    hi_pad = jnp.concatenate([hi, jnp.zeros((pad,), jnp.int32)])
    ids = jnp.arange(P, dtype=jnp.int32)

    t3 = t_pad.reshape(NW, NK, KB)
    ids3 = ids.reshape(NW, NK, KB)
    lo3 = lo_pad.reshape(NW, NK, KB)
    hi3 = hi_pad.reshape(NW, NK, KB)
    lohi = jnp.concatenate([lo3, hi3], axis=1)


    enW1f = p['enW1'] / p['node_std'][:, None]
    enb1f = p['enb1'] - jnp.dot(p['node_mean'] / p['node_std'], p['enW1'])
    eeW1f = p['eeW1'] / p['edge_std'][:, None]
    eeb1f = p['eeb1'] - jnp.dot(p['edge_mean'] / p['edge_std'], p['eeW1'])
    eeWr = eeW1f[0:2]
    eeWn = eeW1f[2]

    w0 = p['We1'][0]
    h0, j0 = _tc_call(
        _tc1_body, N // RN,
        [_rows((RN, 1)), _rows((RN, 2)), _rows((RN, 2)),
         _full((NTS + 2, L)), _full((1, L)), _full((L, L)), _full((1, L)),
         _full((L, L)), _full((L, L))],
        [_rows((RN, L)), _rows((RN, JW))],
        [jax.ShapeDtypeStruct((N, L), f32), jax.ShapeDtypeStruct((N, JW), f32)],
    )(node_type, velocity, mesh_pos,
      enW1f, _r2(enb1f), p['enW2'], _r2(p['enb2']),
      w0[0:L], w0[L:2 * L])

    table, jg0 = _sc_scatter_gather_j0(t3, ids3, j0, lohi)
    r1, r2 = _sc_mark_reps(t3, lo3, hi3, table)
    recvg = jnp.concatenate([r1.reshape(P), r2.reshape(P)])
    lc1 = recvg - HALF
    recvs = jnp.stack([
        jnp.where(recvg < HALF, recvg, HALF).reshape(16, NKS, KB),
        jnp.where((lc1 >= 0) & (lc1 < HALF), lc1, HALF).reshape(16, NKS, KB)])

    he1 = _tc_call(
        _tc2_body, P // RE,
        [_half3((1, RE, JW), 0), _half3((1, RE, JW), 1),
         _full((2, L)), _full((1, L)), _full((1, L)),
         _full((L, L)), _full((1, L)), _full((L, L)), _full((1, L)),
         _full((L, L)), _full((1, L))],
        _rows3((2, RE, L)),
        jax.ShapeDtypeStruct((2, P, L), f32),
    )(jg0, jg0, eeWr, _r2(eeWn), _r2(eeb1f), p['eeW2'], _r2(p['eeb2']),
      w0[2 * L:3 * L], _r2(p['be1'][0]), p['We2'][0], _r2(p['be2'][0]))

    zeros = jnp.zeros((ACC_ROWS, L), f32)

    agg0 = _sc_segsum(he1.reshape(2 * P, L), recvs, zeros)

    wn0 = p['Wn1'][0]
    w1 = p['We1'][1]
    h1, j1 = _tc_call(
        _tc3_body, N // RN,
        [_rows((RN, L)), _aggspec(),
         _full((L, L)), _full((L, L)), _full((1, L)), _full((L, L)),
         _full((1, L)), _full((L, L)), _full((L, L))],
        [_rows((RN, L)), _rows((RN, JW1))],
        [jax.ShapeDtypeStruct((N, L), f32), jax.ShapeDtypeStruct((N, JW1), f32)],
    )(h0, agg0, wn0[0:L], wn0[L:2 * L], _r2(p['bn1'][0]), p['Wn2'][0],
      _r2(p['bn2'][0]), w1[0:L], w1[L:2 * L])

    jg1 = _sc_gather_j1(j1, lohi)

    he2 = _tc_call(
        _tc4_body, P // RE,
        [_half3((1, RE, JW1), 0), _half3((1, RE, JW1), 1), _rows3((2, RE, L)),
         _full((L, L)), _full((1, L)), _full((L, L)), _full((1, L))],
        _rows3((2, RE, L)),
        jax.ShapeDtypeStruct((2, P, L), f32),
    )(jg1, jg1, he1, w1[2 * L:3 * L], _r2(p['be1'][1]), p['We2'][1],
      _r2(p['be2'][1]))

    agg1 = _sc_segsum(he2.reshape(2 * P, L), recvs, zeros)

    wn1 = p['Wn1'][1]
    out, updated = _tc_call(
        _tc5_body, N // RN,
        [_rows((RN, L)), _aggspec(), _rows((RN, 2)),
         _full((L, L)), _full((L, L)), _full((1, L)), _full((L, L)),
         _full((1, L)), _full((L, L)), _full((1, L)), _full((L, 2)),
         _full((1, 2)), _full((1, 2)), _full((1, 2))],
        [_rows((RN, 2)), _rows((RN, 2))],
        [jax.ShapeDtypeStruct((N, 2), f32), jax.ShapeDtypeStruct((N, 2), f32)],
    )(h1, agg1, velocity, wn1[0:L], wn1[L:2 * L], _r2(p['bn1'][1]),
      p['Wn2'][1], _r2(p['bn2'][1]), p['dW1'], _r2(p['db1']), p['dW2'],
      _r2(p['db2']), _r2(p['out_mean']), _r2(p['out_std']))

    return jnp.where(is_training != 0, out, updated)


# speedup vs baseline: 1.1132x; 1.0002x over previous
import functools

import jax
import jax.numpy as jnp
from jax import lax
from jax.experimental import pallas as pl
from jax.experimental.pallas import tpu as pltpu
from jax.experimental.pallas import tpu_sc as plsc

N = 50000
NTS = 9
L = 32
E = 300000
NW = 32
CHUNK = 9600
P = NW * CHUNK
KB = 128
NK = CHUNK // KB
NK2 = 2 * NK
BF = 15
TAB = N * (N + 1) // 2
TSIZE = TAB + 8
DUMMY = N
JW = 80
JW1 = 64
SUB = 5
RB = SUB * KB
NBIG = NK // SUB
SUBJ = 3
SGR = 5
EPT = 2 * P // 16
NKS = EPT // KB
NBIGS = NKS // SGR
HALF = N // 2
ACC_ROWS = 25024
STRIPE = ACC_ROWS // 16


def _wid():
    return lax.axis_index("s") * 2 + lax.axis_index("c")


def _lazy(builder):
    cache = []

    def call(*args):
        if not cache:
            cache.append(builder())
        return cache[0](*args)

    return call


def _sc_params():
    return pltpu.CompilerParams(use_tc_tiling_on_sc=False)


def _vmesh():
    return plsc.VectorSubcoreMesh(core_axis_name="c", subcore_axis_name="s")


def _build_sc_scatter_gather_j0():
  @functools.partial(
      pl.kernel,
      out_type=[
          jax.ShapeDtypeStruct((TSIZE,), jnp.int32),
          jax.ShapeDtypeStruct((2, P, JW), jnp.float32),
      ],
      mesh=_vmesh(),
      name="sc_dedup_scatter_gather_j0",
      compiler_params=_sc_params(),
      scratch_types=[
          pltpu.VMEM((NK, KB), jnp.int32),
          pltpu.VMEM((NK, KB), jnp.int32),
          pltpu.VMEM((NK2, KB), jnp.int32),
          pltpu.VMEM((2, SUBJ * KB, JW), jnp.float32),
          pltpu.SemaphoreType.DMA,
          pltpu.SemaphoreType.DMA,
          pltpu.SemaphoreType.DMA,
      ],
  )
  def _fused(t_hbm, ids_hbm, j_hbm, lohi_hbm, table_hbm, out_hbm,
             tv, val_v, idx_v, buf, semt, semg, semw):
    w = _wid()
    pltpu.sync_copy(t_hbm.at[w], tv)
    pltpu.sync_copy(ids_hbm.at[w], val_v)
    pltpu.sync_copy(lohi_hbm.at[w], idx_v)

    def fire_scat(j, c):
        @pl.when(j < NK)
        def _():
            pltpu.async_copy(val_v.at[j], table_hbm.at[tv.at[j]], semt)
        return c

    def drain_scat(j, c):
        pltpu.make_async_copy(val_v.at[0], table_hbm.at[tv.at[0]],
                              semt).wait()
        return c

    NBJ = NK // SUBJ
    SPG = (NK + 2 * NBJ - 1) // (2 * NBJ)

    def body(g, c):
        slot = g % 2

        @pl.when(g >= 2)
        def _():
            pltpu.make_async_copy(
                buf.at[0], out_hbm.at[0].at[pl.ds(0, SUBJ * KB)], semw).wait()

        lax.fori_loop(g * SPG, (g + 1) * SPG, fire_scat, 0)
        h = g // NBJ
        b = g - h * NBJ
        for u in range(SUBJ):
            pltpu.async_copy(
                j_hbm.at[idx_v.at[h * NK + b * SUBJ + u]],
                buf.at[slot].at[pl.ds(u * KB, KB)], semg)
        for u in range(SUBJ):
            pltpu.make_async_copy(
                j_hbm.at[idx_v.at[0]],
                buf.at[0].at[pl.ds(0, KB)], semg).wait()
        row0 = (w * NK + b * SUBJ) * KB
        pltpu.async_copy(buf.at[slot],
                         out_hbm.at[h].at[pl.ds(row0, SUBJ * KB)], semw)
        return c

    lax.fori_loop(0, 2 * NBJ, body, 0)
    for _ in range(2):
        pltpu.make_async_copy(
            buf.at[0], out_hbm.at[0].at[pl.ds(0, SUBJ * KB)], semw).wait()
    lax.fori_loop(0, NK, drain_scat, 0)

  return _fused


_sc_scatter_gather_j0 = _lazy(_build_sc_scatter_gather_j0)


def _build_sc_mark_reps():
  @functools.partial(
      pl.kernel,
      out_type=[
          jax.ShapeDtypeStruct((NW, NK, KB), jnp.int32),
          jax.ShapeDtypeStruct((NW, NK, KB), jnp.int32),
      ],
      mesh=_vmesh(),
      name="sc_dedup_mark",
      compiler_params=_sc_params(),
      scratch_types=[
          pltpu.VMEM((NK, KB), jnp.int32),
          pltpu.VMEM((NK, KB), jnp.int32),
          pltpu.VMEM((NK, KB), jnp.int32),
          pltpu.VMEM((NK, KB), jnp.int32),
          pltpu.VMEM((NK, KB), jnp.int32),
          pltpu.VMEM((NK, KB), jnp.int32),
          pltpu.SemaphoreType.DMA,
      ],
  )
  def _sc_mark_reps(t_hbm, lo_hbm, hi_hbm, table_hbm, r1_hbm, r2_hbm,
                    idx_v, w_v, lo_v, hi_v, r1_v, r2_v, sem):
    w = _wid()
    pltpu.sync_copy(t_hbm.at[w], idx_v)
    pltpu.sync_copy(lo_hbm.at[w], lo_v)
    pltpu.sync_copy(hi_hbm.at[w], hi_v)

    def fire(j, c):
        pltpu.async_copy(table_hbm.at[idx_v.at[j]], w_v.at[j], sem)
        return c

    def drain(j, c):
        pltpu.make_async_copy(table_hbm.at[idx_v.at[0]], w_v.at[0],
                              sem).wait()
        return c

    def batch(b, c):
        lax.fori_loop(b * BF, (b + 1) * BF, fire, 0)
        lax.fori_loop(0, BF, drain, 0)
        return c

    lax.fori_loop(0, NK // BF, batch, 0)

    base = w * CHUNK
    lanes = lax.broadcasted_iota(jnp.int32, (16,), 0)

    def cbody(i, carry):
        j = i // 8
        o = (i % 8) * 16
        wv = w_v[j, pl.ds(o, 16)]
        lov = lo_v[j, pl.ds(o, 16)]
        hiv = hi_v[j, pl.ds(o, 16)]
        pos = base + i * 16 + lanes
        m = (wv == pos) & (pos < E)
        r1_v[j, pl.ds(o, 16)] = jnp.where(m, hiv, DUMMY)
        r2_v[j, pl.ds(o, 16)] = jnp.where(m, lov, DUMMY)
        return carry

    lax.fori_loop(0, CHUNK // 16, cbody, 0)
    pltpu.sync_copy(r1_v, r1_hbm.at[w])
    pltpu.sync_copy(r2_v, r2_hbm.at[w])

  return _sc_mark_reps


_sc_mark_reps = _lazy(_build_sc_mark_reps)


def _make_sc_gather(width, name):
    @functools.partial(
        pl.kernel,
        out_type=jax.ShapeDtypeStruct((2, P, width), jnp.float32),
        mesh=_vmesh(),
        name=name,
        compiler_params=_sc_params(),
        scratch_types=[
            pltpu.VMEM((NK2, KB), jnp.int32),
            pltpu.VMEM((2, RB, width), jnp.float32),
            pltpu.SemaphoreType.DMA,
            pltpu.SemaphoreType.DMA,
        ],
    )
    def _sc_gather(j_hbm, idx_hbm, out_hbm, idx_v, buf, semg, semw):
        w = _wid()
        pltpu.sync_copy(idx_hbm.at[w], idx_v)

        def body(g, c):
            slot = g % 2

            @pl.when(g >= 2)
            def _():
                pltpu.make_async_copy(
                    buf.at[0], out_hbm.at[0].at[pl.ds(0, RB)], semw).wait()

            h = g // NBIG
            b = g - h * NBIG
            for s in range(SUB):
                pltpu.async_copy(
                    j_hbm.at[idx_v.at[h * NK + b * SUB + s]],
                    buf.at[slot].at[pl.ds(s * KB, KB)], semg)
            for s in range(SUB):
                pltpu.make_async_copy(
                    j_hbm.at[idx_v.at[0]],
                    buf.at[0].at[pl.ds(0, KB)], semg).wait()
            row0 = (w * NK + b * SUB) * KB
            pltpu.async_copy(buf.at[slot],
                             out_hbm.at[h].at[pl.ds(row0, RB)], semw)
            return c

        lax.fori_loop(0, 2 * NBIG, body, 0)
        for _ in range(2):
            pltpu.make_async_copy(
                buf.at[0], out_hbm.at[0].at[pl.ds(0, RB)], semw).wait()

    return _sc_gather


_sc_gather_j1 = _lazy(lambda: _make_sc_gather(JW1, "sc_gather_j1"))


def _build_sc_segsum():
  @functools.partial(
      pl.kernel,
      out_type=jax.ShapeDtypeStruct((2, ACC_ROWS, L), jnp.float32),
      mesh=_vmesh(),
      name="sc_segsum",
      compiler_params=_sc_params(),
      scratch_types=[
          pltpu.VMEM((NKS, KB), jnp.int32),
          pltpu.VMEM((2, SGR * KB, L), jnp.float32),
          pltpu.VMEM_SHARED((ACC_ROWS, L), jnp.float32),
          pltpu.SemaphoreType.DMA,
          pltpu.SemaphoreType.DMA,
          pltpu.SemaphoreType.DMA,
      ],
  )
  def _sc_segsum(he_hbm, recv_hbm, zeros_hbm, agg_hbm, idx_v, buf, acc,
                 seml, sems0, sems1):
    c = lax.axis_index("c")
    s = lax.axis_index("s")
    pltpu.sync_copy(recv_hbm.at[c].at[s], idx_v)
    pltpu.sync_copy(zeros_hbm.at[pl.ds(s * STRIPE, STRIPE)],
                    acc.at[pl.ds(s * STRIPE, STRIPE)])
    plsc.subcore_barrier()

    d = s // 8
    eb = (s - d * 8) * EPT
    last = P - SGR * KB

    def load(q, slot):
        row0 = jnp.minimum(eb + q * SGR * KB, last)
        pltpu.async_copy(he_hbm.at[d].at[pl.ds(row0, SGR * KB)],
                         buf.at[slot], seml)

    def wait_load():
        pltpu.make_async_copy(he_hbm.at[0].at[pl.ds(0, SGR * KB)], buf.at[0],
                              seml).wait()

    def adds(q, slot, sem):
        for u in range(SGR):
            pltpu.async_copy(buf.at[slot].at[pl.ds(u * KB, KB)],
                             acc.at[idx_v.at[q * SGR + u]], sem, add=True)

    def drain_adds(slot, sem):
        for u in range(SGR):
            pltpu.make_async_copy(buf.at[slot].at[pl.ds(u * KB, KB)],
                                  acc.at[pl.ds(0, KB)], sem).wait()

    load(0, 0)

    def body(qq, cc):
        wait_load()

        @pl.when(qq >= 1)
        def _():
            drain_adds(1, sems1)

        load(2 * qq + 1, 1)
        adds(2 * qq, 0, sems0)
        wait_load()
        drain_adds(0, sems0)
        load(2 * qq + 2, 0)
        adds(2 * qq + 1, 1, sems1)
        return cc

    lax.fori_loop(0, NBIGS // 2, body, 0)
    wait_load()
    drain_adds(1, sems1)
    plsc.subcore_barrier()
    pltpu.sync_copy(acc.at[pl.ds(s * STRIPE, STRIPE)],
                    agg_hbm.at[c].at[pl.ds(s * STRIPE, STRIPE)])

  return _sc_segsum


_sc_segsum = _lazy(_build_sc_segsum)


RN = 1000
RE = 4096


def _tc1_body(nt_ref, vel_ref, mp_ref, enW1, enb1, enW2, enb2,
              Wa, Wb, h_ref, j_ref):
    nt = nt_ref[...]
    oh = (lax.broadcasted_iota(jnp.int32, (RN, NTS), 1) == nt).astype(jnp.float32)
    nf = jnp.concatenate([vel_ref[...], oh], axis=1)
    h = jnp.dot(jax.nn.relu(jnp.dot(nf, enW1[...]) + enb1[...]),
                enW2[...]) + enb2[...]
    h_ref[...] = h
    z = jnp.zeros((RN, JW - 66), jnp.float32)
    j_ref[...] = jnp.concatenate(
        [jnp.dot(h, Wa[...]), jnp.dot(h, Wb[...]), mp_ref[...], z], axis=1)


def _tc2_body(jg_ref, eeWr, eeWn, eeb1, eeW2, eeb2,
              Wc, b1, W2, b2, he_ref):
    jlo = jg_ref[0]
    jhi = jg_ref[1]
    rel = jlo[:, 64:66] - jhi[:, 64:66]
    nrm = jnp.sqrt(jnp.sum(rel * rel, axis=1, keepdims=True))
    relW = jnp.dot(rel, eeWr[...])
    base = jnp.dot(nrm, eeWn[...]) + eeb1[...]
    for d in (0, 1):
        he0 = jnp.dot(jax.nn.relu(base + relW if d == 0 else base - relW),
                      eeW2[...]) + eeb2[...]
        if d == 0:
            pre = jlo[:, 0:32] + jhi[:, 32:64] + jnp.dot(he0, Wc[...]) + b1[...]
        else:
            pre = jhi[:, 0:32] + jlo[:, 32:64] + jnp.dot(he0, Wc[...]) + b1[...]
        he_ref[d] = he0 + jnp.dot(jax.nn.relu(pre), W2[...]) + b2[...]


def _tc3_body(h_ref, agg_ref, Wh, Wg, bn1, Wn2, bn2, Wa, Wb, h1_ref, j_ref):
    h = h_ref[...]
    agg = agg_ref[0]
    pre = jnp.dot(h, Wh[...]) + jnp.dot(agg, Wg[...]) + bn1[...]
    h1 = h + jnp.dot(jax.nn.relu(pre), Wn2[...]) + bn2[...]
    h1_ref[...] = h1
    j_ref[...] = jnp.concatenate([jnp.dot(h1, Wa[...]), jnp.dot(h1, Wb[...])],
                                 axis=1)


def _tc4_body(jg_ref, he_ref, Wc, b1, W2, b2, heo_ref):
    jlo = jg_ref[0]
    jhi = jg_ref[1]
    for d in (0, 1):
        he = he_ref[d]
        if d == 0:
            pre = jlo[:, 0:32] + jhi[:, 32:64] + jnp.dot(he, Wc[...]) + b1[...]
        else:
            pre = jhi[:, 0:32] + jlo[:, 32:64] + jnp.dot(he, Wc[...]) + b1[...]
        heo_ref[d] = he + jnp.dot(jax.nn.relu(pre), W2[...]) + b2[...]


def _tc5_body(h_ref, agg_ref, vel_ref, Wh, Wg, bn1, Wn2, bn2,
              dW1, db1, dW2, db2, omean, ostd, out_ref, upd_ref):
    h = h_ref[...]
    agg = agg_ref[0]
    pre = jnp.dot(h, Wh[...]) + jnp.dot(agg, Wg[...]) + bn1[...]
    h2 = h + jnp.dot(jax.nn.relu(pre), Wn2[...]) + bn2[...]
    o = jnp.dot(jax.nn.relu(jnp.dot(h2, dW1[...]) + db1[...]), dW2[...]) + db2[...]
    out_ref[...] = o
    upd_ref[...] = vel_ref[...] + o * ostd[...] + omean[...]


def _full(shape):
    return pl.BlockSpec(shape, lambda i: tuple(0 for _ in shape))


def _rows(shape):
    return pl.BlockSpec(shape, lambda i: (i,) + tuple(0 for _ in shape[1:]))


def _rows3(shape):
    return pl.BlockSpec(shape, lambda i: (0, i, 0))


def _half3(shape, h):
    return pl.BlockSpec(shape, lambda i, _h=h: (_h, i, 0))


def _aggspec():
    return pl.BlockSpec((1, RN, L), lambda i: (i // (HALF // RN),
                                               i % (HALF // RN), 0))


def _tc_call(body, grid, in_specs, out_specs, out_shape):
    return pl.pallas_call(body, grid=(grid,), in_specs=in_specs,
                          out_specs=out_specs, out_shape=out_shape)


def _r2(v):
    return v.reshape(1, -1)


def kernel(node_type, velocity, mesh_pos, cells, is_training, params):
    p = params
    f32 = jnp.float32

    e = jnp.concatenate([cells[:, 0:2], cells[:, 1:3],
                         jnp.stack([cells[:, 2], cells[:, 0]], axis=1)], axis=0)
    lo = jnp.minimum(e[:, 0], e[:, 1])
    hi = jnp.maximum(e[:, 0], e[:, 1])
    tri = jnp.where(lo % 2 == 0, (lo // 2) * (lo + 1), lo * ((lo + 1) // 2))
    t = lo * N - tri + hi
    pad = P - E
    t_pad = jnp.concatenate([t, jnp.full((pad,), TAB, jnp.int32)])
    lo_pad = jnp.concatenate([lo, jnp.zeros((pad,), jnp.int32)])
    hi_pad = jnp.concatenate([hi, jnp.zeros((pad,), jnp.int32)])
    ids = jnp.arange(P, dtype=jnp.int32)

    t3 = t_pad.reshape(NW, NK, KB)
    ids3 = ids.reshape(NW, NK, KB)
    lo3 = lo_pad.reshape(NW, NK, KB)
    hi3 = hi_pad.reshape(NW, NK, KB)
    lohi = jnp.concatenate([lo3, hi3], axis=1)


    enW1f = p['enW1'] / p['node_std'][:, None]
    enb1f = p['enb1'] - jnp.dot(p['node_mean'] / p['node_std'], p['enW1'])
    eeW1f = p['eeW1'] / p['edge_std'][:, None]
    eeb1f = p['eeb1'] - jnp.dot(p['edge_mean'] / p['edge_std'], p['eeW1'])
    eeWr = eeW1f[0:2]
    eeWn = eeW1f[2]

    w0 = p['We1'][0]
    h0, j0 = _tc_call(
        _tc1_body, N // RN,
        [_rows((RN, 1)), _rows((RN, 2)), _rows((RN, 2)),
         _full((NTS + 2, L)), _full((1, L)), _full((L, L)), _full((1, L)),
         _full((L, L)), _full((L, L))],
        [_rows((RN, L)), _rows((RN, JW))],
        [jax.ShapeDtypeStruct((N, L), f32), jax.ShapeDtypeStruct((N, JW), f32)],
    )(node_type, velocity, mesh_pos,
      enW1f, _r2(enb1f), p['enW2'], _r2(p['enb2']),
      w0[0:L], w0[L:2 * L])

    table, jg0 = _sc_scatter_gather_j0(t3, ids3, j0, lohi)
    r1, r2 = _sc_mark_reps(t3, lo3, hi3, table)
    recvg = jnp.concatenate([r1.reshape(P), r2.reshape(P)])
    lc1 = recvg - HALF
    recvs = jnp.stack([
        jnp.where(recvg < HALF, recvg, HALF).reshape(16, NKS, KB),
        jnp.where((lc1 >= 0) & (lc1 < HALF), lc1, HALF).reshape(16, NKS, KB)])

    he1 = _tc_call(
        _tc2_body, P // RE,
        [_rows3((2, RE, JW)),
         _full((2, L)), _full((1, L)), _full((1, L)),
         _full((L, L)), _full((1, L)), _full((L, L)), _full((1, L)),
         _full((L, L)), _full((1, L))],
        _rows3((2, RE, L)),
        jax.ShapeDtypeStruct((2, P, L), f32),
    )(jg0, eeWr, _r2(eeWn), _r2(eeb1f), p['eeW2'], _r2(p['eeb2']),
      w0[2 * L:3 * L], _r2(p['be1'][0]), p['We2'][0], _r2(p['be2'][0]))

    zeros = jnp.zeros((ACC_ROWS, L), f32)

    agg0 = _sc_segsum(he1, recvs, zeros)

    wn0 = p['Wn1'][0]
    w1 = p['We1'][1]
    h1, j1 = _tc_call(
        _tc3_body, N // RN,
        [_rows((RN, L)), _aggspec(),
         _full((L, L)), _full((L, L)), _full((1, L)), _full((L, L)),
         _full((1, L)), _full((L, L)), _full((L, L))],
        [_rows((RN, L)), _rows((RN, JW1))],
        [jax.ShapeDtypeStruct((N, L), f32), jax.ShapeDtypeStruct((N, JW1), f32)],
    )(h0, agg0, wn0[0:L], wn0[L:2 * L], _r2(p['bn1'][0]), p['Wn2'][0],
      _r2(p['bn2'][0]), w1[0:L], w1[L:2 * L])

    jg1 = _sc_gather_j1(j1, lohi)

    he2 = _tc_call(
        _tc4_body, P // RE,
        [_rows3((2, RE, JW1)), _rows3((2, RE, L)),
         _full((L, L)), _full((1, L)), _full((L, L)), _full((1, L))],
        _rows3((2, RE, L)),
        jax.ShapeDtypeStruct((2, P, L), f32),
    )(jg1, he1, w1[2 * L:3 * L], _r2(p['be1'][1]), p['We2'][1],
      _r2(p['be2'][1]))

    agg1 = _sc_segsum(he2, recvs, zeros)

    wn1 = p['Wn1'][1]
    out, updated = _tc_call(
        _tc5_body, N // RN,
        [_rows((RN, L)), _aggspec(), _rows((RN, 2)),
         _full((L, L)), _full((L, L)), _full((1, L)), _full((L, L)),
         _full((1, L)), _full((L, L)), _full((1, L)), _full((L, 2)),
         _full((1, 2)), _full((1, 2)), _full((1, 2))],
        [_rows((RN, 2)), _rows((RN, 2))],
        [jax.ShapeDtypeStruct((N, 2), f32), jax.ShapeDtypeStruct((N, 2), f32)],
    )(h1, agg1, velocity, wn1[0:L], wn1[L:2 * L], _r2(p['bn1'][1]),
      p['Wn2'][1], _r2(p['bn2'][1]), p['dW1'], _r2(p['db1']), p['dW2'],
      _r2(p['db2']), _r2(p['out_mean']), _r2(p['out_std']))

    return jnp.where(is_training != 0, out, updated)


# speedup vs baseline: 1.1529x; 1.0357x over previous
import functools

import jax
import jax.numpy as jnp
from jax import lax
from jax.experimental import pallas as pl
from jax.experimental.pallas import tpu as pltpu
from jax.experimental.pallas import tpu_sc as plsc

N = 50000
NTS = 9
L = 32
E = 300000
NW = 32
CHUNK = 9600
P = NW * CHUNK
KB = 128
NK = CHUNK // KB
NK2 = 2 * NK
BF = 15
TAB = N * (N + 1) // 2
TSIZE = TAB + 8
DUMMY = N
JW = 48
JW1 = 32
SUB = 5
RB = SUB * KB
NBIG = NK // SUB
SUBJ = 5
SGR = 5
EPT = 2 * P // 16
NKS = EPT // KB
NBIGS = NKS // SGR
HALF = N // 2
ACC_ROWS = 25024
STRIPE = ACC_ROWS // 16


def _wid():
    return lax.axis_index("s") * 2 + lax.axis_index("c")


def _lazy(builder):
    cache = []

    def call(*args):
        if not cache:
            cache.append(builder())
        return cache[0](*args)

    return call


def _sc_params():
    return pltpu.CompilerParams(use_tc_tiling_on_sc=False)


def _vmesh():
    return plsc.VectorSubcoreMesh(core_axis_name="c", subcore_axis_name="s")


def _build_sc_scatter_gather_j0():
  @functools.partial(
      pl.kernel,
      out_type=[
          jax.ShapeDtypeStruct((TSIZE,), jnp.int32),
          jax.ShapeDtypeStruct((2, P, JW), jnp.float32),
      ],
      mesh=_vmesh(),
      name="sc_dedup_scatter_gather_j0",
      compiler_params=_sc_params(),
      scratch_types=[
          pltpu.VMEM((NK, KB), jnp.int32),
          pltpu.VMEM((NK, KB), jnp.int32),
          pltpu.VMEM((NK2, KB), jnp.int32),
          pltpu.VMEM((2, SUBJ * KB, JW), jnp.float32),
          pltpu.SemaphoreType.DMA,
          pltpu.SemaphoreType.DMA,
          pltpu.SemaphoreType.DMA,
      ],
  )
  def _fused(t_hbm, ids_hbm, j_hbm, lohi_hbm, table_hbm, out_hbm,
             tv, val_v, idx_v, buf, semt, semg, semw):
    w = _wid()
    pltpu.sync_copy(t_hbm.at[w], tv)
    pltpu.sync_copy(ids_hbm.at[w], val_v)
    pltpu.sync_copy(lohi_hbm.at[w], idx_v)

    def fire_scat(j, c):
        @pl.when(j < NK)
        def _():
            pltpu.async_copy(val_v.at[j], table_hbm.at[tv.at[j]], semt)
        return c

    def drain_scat(j, c):
        pltpu.make_async_copy(val_v.at[0], table_hbm.at[tv.at[0]],
                              semt).wait()
        return c

    NBJ = NK // SUBJ
    SPG = (NK + 2 * NBJ - 1) // (2 * NBJ)

    def body(g, c):
        slot = g % 2

        @pl.when(g >= 2)
        def _():
            pltpu.make_async_copy(
                buf.at[0], out_hbm.at[0].at[pl.ds(0, SUBJ * KB)], semw).wait()

        lax.fori_loop(g * SPG, (g + 1) * SPG, fire_scat, 0)
        h = g // NBJ
        b = g - h * NBJ
        for u in range(SUBJ):
            pltpu.async_copy(
                j_hbm.at[idx_v.at[h * NK + b * SUBJ + u]],
                buf.at[slot].at[pl.ds(u * KB, KB)], semg)
        for u in range(SUBJ):
            pltpu.make_async_copy(
                j_hbm.at[idx_v.at[0]],
                buf.at[0].at[pl.ds(0, KB)], semg).wait()
        row0 = (w * NK + b * SUBJ) * KB
        pltpu.async_copy(buf.at[slot],
                         out_hbm.at[h].at[pl.ds(row0, SUBJ * KB)], semw)
        return c

    lax.fori_loop(0, 2 * NBJ, body, 0)
    for _ in range(2):
        pltpu.make_async_copy(
            buf.at[0], out_hbm.at[0].at[pl.ds(0, SUBJ * KB)], semw).wait()
    lax.fori_loop(0, NK, drain_scat, 0)

  return _fused


_sc_scatter_gather_j0 = _lazy(_build_sc_scatter_gather_j0)


def _build_sc_mark_reps():
  @functools.partial(
      pl.kernel,
      out_type=[
          jax.ShapeDtypeStruct((NW, NK, KB), jnp.int32),
          jax.ShapeDtypeStruct((NW, NK, KB), jnp.int32),
      ],
      mesh=_vmesh(),
      name="sc_dedup_mark",
      compiler_params=_sc_params(),
      scratch_types=[
          pltpu.VMEM((NK, KB), jnp.int32),
          pltpu.VMEM((NK, KB), jnp.int32),
          pltpu.VMEM((NK, KB), jnp.int32),
          pltpu.VMEM((NK, KB), jnp.int32),
          pltpu.VMEM((NK, KB), jnp.int32),
          pltpu.VMEM((NK, KB), jnp.int32),
          pltpu.SemaphoreType.DMA,
      ],
  )
  def _sc_mark_reps(t_hbm, lo_hbm, hi_hbm, table_hbm, r1_hbm, r2_hbm,
                    idx_v, w_v, lo_v, hi_v, r1_v, r2_v, sem):
    w = _wid()
    pltpu.sync_copy(t_hbm.at[w], idx_v)
    pltpu.sync_copy(lo_hbm.at[w], lo_v)
    pltpu.sync_copy(hi_hbm.at[w], hi_v)

    def fire(j, c):
        pltpu.async_copy(table_hbm.at[idx_v.at[j]], w_v.at[j], sem)
        return c

    def drain(j, c):
        pltpu.make_async_copy(table_hbm.at[idx_v.at[0]], w_v.at[0],
                              sem).wait()
        return c

    def batch(b, c):
        lax.fori_loop(b * BF, (b + 1) * BF, fire, 0)
        lax.fori_loop(0, BF, drain, 0)
        return c

    lax.fori_loop(0, NK // BF, batch, 0)

    base = w * CHUNK
    lanes = lax.broadcasted_iota(jnp.int32, (16,), 0)

    def cbody(i, carry):
        j = i // 8
        o = (i % 8) * 16
        wv = w_v[j, pl.ds(o, 16)]
        lov = lo_v[j, pl.ds(o, 16)]
        hiv = hi_v[j, pl.ds(o, 16)]
        pos = base + i * 16 + lanes
        m = (wv == pos) & (pos < E)
        r1_v[j, pl.ds(o, 16)] = jnp.where(m, hiv, DUMMY)
        r2_v[j, pl.ds(o, 16)] = jnp.where(m, lov, DUMMY)
        return carry

    lax.fori_loop(0, CHUNK // 16, cbody, 0)
    pltpu.sync_copy(r1_v, r1_hbm.at[w])
    pltpu.sync_copy(r2_v, r2_hbm.at[w])

  return _sc_mark_reps


_sc_mark_reps = _lazy(_build_sc_mark_reps)


def _make_sc_gather(width, name):
    @functools.partial(
        pl.kernel,
        out_type=jax.ShapeDtypeStruct((2, P, width), jnp.float32),
        mesh=_vmesh(),
        name=name,
        compiler_params=_sc_params(),
        scratch_types=[
            pltpu.VMEM((NK2, KB), jnp.int32),
            pltpu.VMEM((2, RB, width), jnp.float32),
            pltpu.SemaphoreType.DMA,
            pltpu.SemaphoreType.DMA,
        ],
    )
    def _sc_gather(j_hbm, idx_hbm, out_hbm, idx_v, buf, semg, semw):
        w = _wid()
        pltpu.sync_copy(idx_hbm.at[w], idx_v)

        def body(g, c):
            slot = g % 2

            @pl.when(g >= 2)
            def _():
                pltpu.make_async_copy(
                    buf.at[0], out_hbm.at[0].at[pl.ds(0, RB)], semw).wait()

            h = g // NBIG
            b = g - h * NBIG
            for s in range(SUB):
                pltpu.async_copy(
                    j_hbm.at[idx_v.at[h * NK + b * SUB + s]],
                    buf.at[slot].at[pl.ds(s * KB, KB)], semg)
            for s in range(SUB):
                pltpu.make_async_copy(
                    j_hbm.at[idx_v.at[0]],
                    buf.at[0].at[pl.ds(0, KB)], semg).wait()
            row0 = (w * NK + b * SUB) * KB
            pltpu.async_copy(buf.at[slot],
                             out_hbm.at[h].at[pl.ds(row0, RB)], semw)
            return c

        lax.fori_loop(0, 2 * NBIG, body, 0)
        for _ in range(2):
            pltpu.make_async_copy(
                buf.at[0], out_hbm.at[0].at[pl.ds(0, RB)], semw).wait()

    return _sc_gather


_sc_gather_j1 = _lazy(lambda: _make_sc_gather(JW1, "sc_gather_j1"))


def _build_sc_segsum():
  @functools.partial(
      pl.kernel,
      out_type=jax.ShapeDtypeStruct((2, ACC_ROWS, L), jnp.float32),
      mesh=_vmesh(),
      name="sc_segsum",
      compiler_params=_sc_params(),
      scratch_types=[
          pltpu.VMEM((NKS, KB), jnp.int32),
          pltpu.VMEM((2, SGR * KB, L), jnp.float32),
          pltpu.VMEM_SHARED((ACC_ROWS, L), jnp.float32),
          pltpu.SemaphoreType.DMA,
          pltpu.SemaphoreType.DMA,
          pltpu.SemaphoreType.DMA,
      ],
  )
  def _sc_segsum(he_hbm, recv_hbm, zeros_hbm, agg_hbm, idx_v, buf, acc,
                 seml, sems0, sems1):
    c = lax.axis_index("c")
    s = lax.axis_index("s")
    pltpu.sync_copy(recv_hbm.at[c].at[s], idx_v)
    pltpu.sync_copy(zeros_hbm.at[pl.ds(s * STRIPE, STRIPE)],
                    acc.at[pl.ds(s * STRIPE, STRIPE)])
    plsc.subcore_barrier()

    d = s // 8
    eb = (s - d * 8) * EPT
    last = P - SGR * KB

    def load(q, slot):
        row0 = jnp.minimum(eb + q * SGR * KB, last)
        pltpu.async_copy(he_hbm.at[d].at[pl.ds(row0, SGR * KB)],
                         buf.at[slot], seml)

    def wait_load():
        pltpu.make_async_copy(he_hbm.at[0].at[pl.ds(0, SGR * KB)], buf.at[0],
                              seml).wait()

    def adds(q, slot, sem):
        for u in range(SGR):
            pltpu.async_copy(buf.at[slot].at[pl.ds(u * KB, KB)],
                             acc.at[idx_v.at[q * SGR + u]], sem, add=True)

    def drain_adds(slot, sem):
        for u in range(SGR):
            pltpu.make_async_copy(buf.at[slot].at[pl.ds(u * KB, KB)],
                                  acc.at[pl.ds(0, KB)], sem).wait()

    load(0, 0)

    def body(qq, cc):
        wait_load()

        @pl.when(qq >= 1)
        def _():
            drain_adds(1, sems1)

        load(2 * qq + 1, 1)
        adds(2 * qq, 0, sems0)
        wait_load()
        drain_adds(0, sems0)
        load(2 * qq + 2, 0)
        adds(2 * qq + 1, 1, sems1)
        return cc

    lax.fori_loop(0, NBIGS // 2, body, 0)
    wait_load()
    drain_adds(1, sems1)
    plsc.subcore_barrier()
    pltpu.sync_copy(acc.at[pl.ds(s * STRIPE, STRIPE)],
                    agg_hbm.at[c].at[pl.ds(s * STRIPE, STRIPE)])

  return _sc_segsum


_sc_segsum = _lazy(_build_sc_segsum)


RN = 1000
RE = 4096


def _tc1_body(nt_ref, vel_ref, mp_ref, enW1, enb1, enW2, enb2,
              h_ref, j_ref):
    nt = nt_ref[...]
    oh = (lax.broadcasted_iota(jnp.int32, (RN, NTS), 1) == nt).astype(jnp.float32)
    nf = jnp.concatenate([vel_ref[...], oh], axis=1)
    h = jnp.dot(jax.nn.relu(jnp.dot(nf, enW1[...]) + enb1[...]),
                enW2[...]) + enb2[...]
    h_ref[...] = h
    z = jnp.zeros((RN, JW - 34), jnp.float32)
    j_ref[...] = jnp.concatenate([h, mp_ref[...], z], axis=1)


def _tc2_body(jg_ref, eeWr, eeWn, eeb1, eeW2, eeb2,
              Wa, Wb, Wc, b1, W2, b2, he_ref):
    hlo = jg_ref[0, :, 0:32]
    hhi = jg_ref[1, :, 0:32]
    rel = jg_ref[0, :, 32:34] - jg_ref[1, :, 32:34]
    nrm = jnp.sqrt(jnp.sum(rel * rel, axis=1, keepdims=True))
    relW = jnp.dot(rel, eeWr[...])
    base = jnp.dot(nrm, eeWn[...]) + eeb1[...]
    alo = jnp.dot(hlo, Wa[...])
    ahi = jnp.dot(hhi, Wa[...])
    blo = jnp.dot(hlo, Wb[...])
    bhi = jnp.dot(hhi, Wb[...])
    for d in (0, 1):
        he0 = jnp.dot(jax.nn.relu(base + relW if d == 0 else base - relW),
                      eeW2[...]) + eeb2[...]
        if d == 0:
            pre = alo + bhi + jnp.dot(he0, Wc[...]) + b1[...]
        else:
            pre = ahi + blo + jnp.dot(he0, Wc[...]) + b1[...]
        he_ref[d] = he0 + jnp.dot(jax.nn.relu(pre), W2[...]) + b2[...]


def _tc3_body(h_ref, agg_ref, Wh, Wg, bn1, Wn2, bn2, h1_ref):
    h = h_ref[...]
    agg = agg_ref[0]
    pre = jnp.dot(h, Wh[...]) + jnp.dot(agg, Wg[...]) + bn1[...]
    h1_ref[...] = h + jnp.dot(jax.nn.relu(pre), Wn2[...]) + bn2[...]


def _tc4_body(jg_ref, he_ref, Wa, Wb, Wc, b1, W2, b2, heo_ref):
    hlo = jg_ref[0]
    hhi = jg_ref[1]
    alo = jnp.dot(hlo, Wa[...])
    ahi = jnp.dot(hhi, Wa[...])
    blo = jnp.dot(hlo, Wb[...])
    bhi = jnp.dot(hhi, Wb[...])
    for d in (0, 1):
        he = he_ref[d]
        if d == 0:
            pre = alo + bhi + jnp.dot(he, Wc[...]) + b1[...]
        else:
            pre = ahi + blo + jnp.dot(he, Wc[...]) + b1[...]
        heo_ref[d] = he + jnp.dot(jax.nn.relu(pre), W2[...]) + b2[...]


def _tc5_body(h_ref, agg_ref, vel_ref, Wh, Wg, bn1, Wn2, bn2,
              dW1, db1, dW2, db2, omean, ostd, out_ref, upd_ref):
    h = h_ref[...]
    agg = agg_ref[0]
    pre = jnp.dot(h, Wh[...]) + jnp.dot(agg, Wg[...]) + bn1[...]
    h2 = h + jnp.dot(jax.nn.relu(pre), Wn2[...]) + bn2[...]
    o = jnp.dot(jax.nn.relu(jnp.dot(h2, dW1[...]) + db1[...]), dW2[...]) + db2[...]
    out_ref[...] = o
    upd_ref[...] = vel_ref[...] + o * ostd[...] + omean[...]


def _full(shape):
    return pl.BlockSpec(shape, lambda i: tuple(0 for _ in shape))


def _rows(shape):
    return pl.BlockSpec(shape, lambda i: (i,) + tuple(0 for _ in shape[1:]))


def _rows3(shape):
    return pl.BlockSpec(shape, lambda i: (0, i, 0))


def _half3(shape, h):
    return pl.BlockSpec(shape, lambda i, _h=h: (_h, i, 0))


def _aggspec():
    return pl.BlockSpec((1, RN, L), lambda i: (i // (HALF // RN),
                                               i % (HALF // RN), 0))


def _tc_call(body, grid, in_specs, out_specs, out_shape):
    return pl.pallas_call(body, grid=(grid,), in_specs=in_specs,
                          out_specs=out_specs, out_shape=out_shape)


def _r2(v):
    return v.reshape(1, -1)


def kernel(node_type, velocity, mesh_pos, cells, is_training, params):
    p = params
    f32 = jnp.float32

    e = jnp.concatenate([cells[:, 0:2], cells[:, 1:3],
                         jnp.stack([cells[:, 2], cells[:, 0]], axis=1)], axis=0)
    lo = jnp.minimum(e[:, 0], e[:, 1])
    hi = jnp.maximum(e[:, 0], e[:, 1])
    tri = jnp.where(lo % 2 == 0, (lo // 2) * (lo + 1), lo * ((lo + 1) // 2))
    t = lo * N - tri + hi
    pad = P - E
    t_pad = jnp.concatenate([t, jnp.full((pad,), TAB, jnp.int32)])
    lo_pad = jnp.concatenate([lo, jnp.zeros((pad,), jnp.int32)])
    hi_pad = jnp.concatenate([hi, jnp.zeros((pad,), jnp.int32)])
    ids = jnp.arange(P, dtype=jnp.int32)

    t3 = t_pad.reshape(NW, NK, KB)
    ids3 = ids.reshape(NW, NK, KB)
    lo3 = lo_pad.reshape(NW, NK, KB)
    hi3 = hi_pad.reshape(NW, NK, KB)
    lohi = jnp.concatenate([lo3, hi3], axis=1)


    enW1f = p['enW1'] / p['node_std'][:, None]
    enb1f = p['enb1'] - jnp.dot(p['node_mean'] / p['node_std'], p['enW1'])
    eeW1f = p['eeW1'] / p['edge_std'][:, None]
    eeb1f = p['eeb1'] - jnp.dot(p['edge_mean'] / p['edge_std'], p['eeW1'])
    eeWr = eeW1f[0:2]
    eeWn = eeW1f[2]

    w0 = p['We1'][0]
    h0, j0 = _tc_call(
        _tc1_body, N // RN,
        [_rows((RN, 1)), _rows((RN, 2)), _rows((RN, 2)),
         _full((NTS + 2, L)), _full((1, L)), _full((L, L)), _full((1, L))],
        [_rows((RN, L)), _rows((RN, JW))],
        [jax.ShapeDtypeStruct((N, L), f32), jax.ShapeDtypeStruct((N, JW), f32)],
    )(node_type, velocity, mesh_pos,
      enW1f, _r2(enb1f), p['enW2'], _r2(p['enb2']))

    table, jg0 = _sc_scatter_gather_j0(t3, ids3, j0, lohi)
    r1, r2 = _sc_mark_reps(t3, lo3, hi3, table)
    recvg = jnp.concatenate([r1.reshape(P), r2.reshape(P)])
    lc1 = recvg - HALF
    recvs = jnp.stack([
        jnp.where(recvg < HALF, recvg, HALF).reshape(16, NKS, KB),
        jnp.where((lc1 >= 0) & (lc1 < HALF), lc1, HALF).reshape(16, NKS, KB)])

    he1 = _tc_call(
        _tc2_body, P // RE,
        [_rows3((2, RE, JW)),
         _full((2, L)), _full((1, L)), _full((1, L)),
         _full((L, L)), _full((1, L)),
         _full((L, L)), _full((L, L)), _full((L, L)),
         _full((1, L)), _full((L, L)), _full((1, L))],
        _rows3((2, RE, L)),
        jax.ShapeDtypeStruct((2, P, L), f32),
    )(jg0, eeWr, _r2(eeWn), _r2(eeb1f), p['eeW2'], _r2(p['eeb2']),
      w0[0:L], w0[L:2 * L], w0[2 * L:3 * L],
      _r2(p['be1'][0]), p['We2'][0], _r2(p['be2'][0]))

    zeros = jnp.zeros((ACC_ROWS, L), f32)

    agg0 = _sc_segsum(he1, recvs, zeros)

    wn0 = p['Wn1'][0]
    w1 = p['We1'][1]
    h1 = _tc_call(
        _tc3_body, N // RN,
        [_rows((RN, L)), _aggspec(),
         _full((L, L)), _full((L, L)), _full((1, L)), _full((L, L)),
         _full((1, L))],
        _rows((RN, L)),
        jax.ShapeDtypeStruct((N, L), f32),
    )(h0, agg0, wn0[0:L], wn0[L:2 * L], _r2(p['bn1'][0]), p['Wn2'][0],
      _r2(p['bn2'][0]))

    jg1 = _sc_gather_j1(h1, lohi)

    he2 = _tc_call(
        _tc4_body, P // RE,
        [_rows3((2, RE, JW1)), _rows3((2, RE, L)),
         _full((L, L)), _full((L, L)), _full((L, L)), _full((1, L)),
         _full((L, L)), _full((1, L))],
        _rows3((2, RE, L)),
        jax.ShapeDtypeStruct((2, P, L), f32),
    )(jg1, he1, w1[0:L], w1[L:2 * L], w1[2 * L:3 * L],
      _r2(p['be1'][1]), p['We2'][1], _r2(p['be2'][1]))

    agg1 = _sc_segsum(he2, recvs, zeros)

    wn1 = p['Wn1'][1]
    out, updated = _tc_call(
        _tc5_body, N // RN,
        [_rows((RN, L)), _aggspec(), _rows((RN, 2)),
         _full((L, L)), _full((L, L)), _full((1, L)), _full((L, L)),
         _full((1, L)), _full((L, L)), _full((1, L)), _full((L, 2)),
         _full((1, 2)), _full((1, 2)), _full((1, 2))],
        [_rows((RN, 2)), _rows((RN, 2))],
        [jax.ShapeDtypeStruct((N, 2), f32), jax.ShapeDtypeStruct((N, 2), f32)],
    )(h1, agg1, velocity, wn1[0:L], wn1[L:2 * L], _r2(p['bn1'][1]),
      p['Wn2'][1], _r2(p['bn2'][1]), p['dW1'], _r2(p['db1']), p['dW2'],
      _r2(p['db2']), _r2(p['out_mean']), _r2(p['out_std']))

    return jnp.where(is_training != 0, out, updated)
